# baseline (speedup 1.0000x reference)
import functools

import jax
import jax.numpy as jnp
from jax import lax
from jax.experimental import pallas as pl
from jax.experimental.pallas import tpu as pltpu
from jax.experimental.pallas import tpu_sc as plsc

_N = 10000
_E = 320000
_D = 128
_OUT = 128
_EPS = 1e-5

_NC = 2
_NS = 16
_NP = 10240
_RPT = _NP // _NS
_CH = 128
_NCH = 160
_NBUF = 4
_G = 16
_HW = _D // _NC // 2
_HALF = _D // _NC
_EPT = _NCH * _CH
_EP = _EPT * _NS

_mesh = plsc.VectorSubcoreMesh(
    core_axis_name="c", subcore_axis_name="s", num_cores=_NC, num_subcores=_NS
)


_DW = 16


@functools.partial(
    pl.kernel,
    out_type=jax.ShapeDtypeStruct((_NC, _NP, _DW), jnp.float32),
    mesh=_mesh,
    scratch_types=[
        pltpu.VMEM((_NCH, _CH), jnp.int32),
        pltpu.VMEM((_CH, _DW), jnp.float32),
        pltpu.VMEM_SHARED((_NP, _DW), jnp.float32),
        pltpu.SemaphoreType.DMA,
    ],
    compiler_params=pltpu.CompilerParams(use_tc_tiling_on_sc=False),
)
def _deg_kernel(idx_hbm, ones_hbm, zeros_hbm, out_hbm, idx_v, ones_v, deg_sp,
                dsem):
    c = lax.axis_index("c")
    s = lax.axis_index("s")
    pltpu.sync_copy(ones_hbm, ones_v)
    pltpu.sync_copy(zeros_hbm, deg_sp.at[pl.ds(s * _RPT, _RPT)])
    pltpu.sync_copy(idx_hbm.at[c, s], idx_v)
    plsc.subcore_barrier()
    def body(j, carry):
        pltpu.async_copy(ones_v, deg_sp.at[idx_v.at[j]], dsem, add=True)
        return carry
    lax.fori_loop(0, _NCH, body, 0)
    def drain(j, carry):
        pltpu.make_async_copy(ones_v, deg_sp.at[idx_v.at[0]], dsem).wait()
        return carry
    lax.fori_loop(0, _NCH, drain, 0)
    plsc.subcore_barrier()
    pltpu.sync_copy(
        deg_sp.at[pl.ds(s * _RPT, _RPT)], out_hbm.at[c, pl.ds(s * _RPT, _RPT)]
    )


@functools.partial(
    pl.kernel,
    out_type=jax.ShapeDtypeStruct((_NC, _NP, _HALF), jnp.float32),
    mesh=_mesh,
    scratch_types=[
        pltpu.VMEM((_G, _CH), jnp.int32),
        pltpu.VMEM((_G, _CH), jnp.int32),
        [pltpu.VMEM((_CH, _HW), jnp.int32) for _ in range(_NBUF)],
        [pltpu.VMEM((_CH, _HALF), jnp.float32) for _ in range(_NBUF)],
        pltpu.VMEM_SHARED((_NP, _HW), jnp.int32),
        pltpu.VMEM_SHARED((_NP, _HALF), jnp.float32),
        pltpu.SemaphoreType.DMA,
        pltpu.SemaphoreType.DMA,
    ],
    compiler_params=pltpu.CompilerParams(
        use_tc_tiling_on_sc=False, needs_layout_passes=False
    ),
)
def _edge_kernel(hs_hbm, idx_hbm, zeros_hbm, out_hbm,
                 src_v, dst_v, pbufs, fbufs, hs_sp, acc_sp, gsem, ssem):
    c = lax.axis_index("c")
    s = lax.axis_index("s")
    r0 = s * _RPT
    pltpu.sync_copy(zeros_hbm, acc_sp.at[pl.ds(r0, _RPT)])
    pltpu.sync_copy(hs_hbm.at[c, pl.ds(r0, _RPT)], hs_sp.at[pl.ds(r0, _RPT)])
    plsc.subcore_barrier()
    mask_hi = jnp.full((16,), -65536, jnp.int32)
    def expand(b):
        def row(rr, carry3):
            for t in range(_HW // 16):
                w = pbufs[b][rr, pl.ds(t * 16, 16)]
                lo = plsc.bitcast(lax.shift_left(w, 16), jnp.float32)
                hi = plsc.bitcast(
                    lax.bitwise_and(w, mask_hi), jnp.float32
                )
                fbufs[b][rr, pl.ds(t * 32, 16)] = lo
                fbufs[b][rr, pl.ds(t * 32 + 16, 16)] = hi
            return carry3
        lax.fori_loop(0, _CH, row, 0)
    def group(g, carry):
        pltpu.sync_copy(idx_hbm.at[0, s, pl.ds(g * _G, _G)], src_v)
        pltpu.sync_copy(idx_hbm.at[1, s, pl.ds(g * _G, _G)], dst_v)
        def round_(r, carry2):
            base = r * _NBUF
            def fire(b):
                pltpu.async_copy(hs_sp.at[src_v.at[base + b]], pbufs[b], gsem)
            @pl.when(r > 0)
            def _():
                for b in range(_NBUF):
                    pltpu.make_async_copy(
                        fbufs[b], acc_sp.at[dst_v.at[b]], ssem
                    ).wait()
                    fire(b)
            @pl.when(r == 0)
            def _():
                for b in range(_NBUF):
                    fire(b)
            for b in range(_NBUF):
                pltpu.make_async_copy(
                    hs_sp.at[src_v.at[base + b]], pbufs[b], gsem
                ).wait()
                expand(b)
                pltpu.async_copy(
                    fbufs[b], acc_sp.at[dst_v.at[base + b]], ssem, add=True
                )
            return carry2
        carry = lax.fori_loop(0, _G // _NBUF, round_, carry)
        for b in range(_NBUF):
            pltpu.make_async_copy(
                fbufs[b], acc_sp.at[dst_v.at[b]], ssem
            ).wait()
        return carry
    lax.fori_loop(0, _NCH // _G, group, 0)
    plsc.subcore_barrier()
    pltpu.sync_copy(
        acc_sp.at[pl.ds(r0, _RPT)],
        out_hbm.at[c, pl.ds(r0, _RPT)],
    )


_BLK1 = 1024


def _ln_body(x_ref, g_ref, b_ref, h_ref):
    xb = x_ref[...]
    mu = jnp.mean(xb, axis=-1, keepdims=True)
    xc = xb - mu
    var = jnp.mean(xc * xc, axis=-1, keepdims=True)
    h_ref[...] = xc * lax.rsqrt(var + _EPS) * g_ref[...] + b_ref[...]


_ln_call = pl.pallas_call(
    _ln_body,
    grid=(_NP // _BLK1,),
    in_specs=[
        pl.BlockSpec((_BLK1, _D), lambda i: (i, 0)),
        pl.BlockSpec((1, _D), lambda i: (0, 0)),
        pl.BlockSpec((1, _D), lambda i: (0, 0)),
    ],
    out_specs=pl.BlockSpec((_BLK1, _D), lambda i: (i, 0)),
    out_shape=jax.ShapeDtypeStruct((_NP, _D), jnp.float32),
)


def _hs_body(h_ref, deg_ref, hs_ref):
    ns = lax.rsqrt(jnp.maximum(deg_ref[0, :, :1], 1.0))
    rows = lax.broadcasted_iota(jnp.int32, (_BLK1, 1), 0) + pl.program_id(0) * _BLK1
    hs = jnp.where(rows < _N, h_ref[...] * ns, 0.0)
    hs_ref[...] = jnp.stack([hs[:, :_HALF], hs[:, _HALF:]], axis=0)


_hs_call = pl.pallas_call(
    _hs_body,
    grid=(_NP // _BLK1,),
    in_specs=[
        pl.BlockSpec((_BLK1, _D), lambda i: (i, 0)),
        pl.BlockSpec((1, _BLK1, _DW), lambda i: (0, i, 0)),
    ],
    out_specs=pl.BlockSpec((_NC, _BLK1, _HALF), lambda i: (0, i, 0)),
    out_shape=jax.ShapeDtypeStruct((_NC, _NP, _HALF), jnp.float32),
)


_BLK2 = 2000


def _ffn_body(h_ref, acc0_ref, acc1_ref, deg_ref, w1_ref, w2_ref, b_ref, o_ref):
    nd = lax.rsqrt(jnp.maximum(deg_ref[0, :, :1], 1.0))
    msg = jnp.concatenate([acc0_ref[...], acc1_ref[...]], axis=1) * nd
    dn = (((1,), (1,)), ((), ()))
    o = lax.dot_general(h_ref[...], w1_ref[...], dn,
                        preferred_element_type=jnp.float32)
    o = o + lax.dot_general(msg, w2_ref[...], dn,
                            preferred_element_type=jnp.float32)
    o_ref[...] = o + b_ref[...]


_ffn_call = pl.pallas_call(
    _ffn_body,
    grid=(_N // _BLK2,),
    in_specs=[
        pl.BlockSpec((_BLK2, _D), lambda i: (i, 0)),
        pl.BlockSpec((_BLK2, _HALF), lambda i: (i, 0)),
        pl.BlockSpec((_BLK2, _HALF), lambda i: (i, 0)),
        pl.BlockSpec((1, _BLK2, _DW), lambda i: (1, i, 0)),
        pl.BlockSpec((_OUT, _D), lambda i: (0, 0)),
        pl.BlockSpec((_OUT, _D), lambda i: (0, 0)),
        pl.BlockSpec((1, _OUT), lambda i: (0, 0)),
    ],
    out_specs=pl.BlockSpec((_BLK2, _OUT), lambda i: (i, 0)),
    out_shape=jax.ShapeDtypeStruct((_N, _OUT), jnp.float32),
)


def _expand_perm():
    p = []
    for t in range(_HALF // 32):
        p += [32 * t + 2 * k for k in range(16)]
        p += [32 * t + 2 * k + 1 for k in range(16)]
    return p


_PERM64 = _expand_perm()
_PERM128 = _PERM64 + [64 + p for p in _PERM64]


def kernel(x, edge_index, gamma, beta, W, b):
    x_pad = jnp.concatenate(
        [x, jnp.zeros((_NP - _N, _D), jnp.float32)], axis=0
    )
    pad = jnp.full((2, _EP - _E), _NP - 1, jnp.int32)
    ei = jnp.concatenate([edge_index, pad], axis=1).reshape(2, _NS, _NCH, _CH)
    deg = _deg_kernel(
        ei,
        jnp.ones((_CH, _DW), jnp.float32),
        jnp.zeros((_RPT, _DW), jnp.float32),
    )
    h = _ln_call(x_pad, gamma.reshape(1, _D), beta.reshape(1, _D))
    hs = _hs_call(h, deg)
    hs_packed = lax.bitcast_convert_type(
        hs.astype(jnp.bfloat16).reshape(_NC, _NP, _HW, 2), jnp.int32
    )
    zeros_tile = jnp.zeros((_RPT, _HALF), jnp.float32)
    acc = _edge_kernel(hs_packed, ei, zeros_tile)
    w2p = W[:, _D:][:, jnp.array(_PERM128, jnp.int32)]
    out = _ffn_call(
        h[:_N], acc[0, :_N], acc[1, :_N], deg, W[:, :_D], w2p,
        b.reshape(1, _OUT),
    )
    return out


# speedup vs baseline: 1.5050x; 1.5050x over previous
import functools

import jax
import jax.numpy as jnp
from jax import lax
from jax.experimental import pallas as pl
from jax.experimental.pallas import tpu as pltpu
from jax.experimental.pallas import tpu_sc as plsc

_N = 10000
_E = 320000
_D = 128
_OUT = 128
_EPS = 1e-5

_NC = 2
_NS = 16
_NP = 10240
_RPT = _NP // _NS
_CH = 128
_NCH = 160
_NBUF = 5
_G = 20
_HALF = _D // _NC
_EPT = _NCH * _CH
_EP = _EPT * _NS

_mesh = plsc.VectorSubcoreMesh(
    core_axis_name="c", subcore_axis_name="s", num_cores=_NC, num_subcores=_NS
)


_DW = 16


@functools.partial(
    pl.kernel,
    out_type=jax.ShapeDtypeStruct((_NC, _NP, _DW), jnp.float32),
    mesh=_mesh,
    scratch_types=[
        pltpu.VMEM((_NCH, _CH), jnp.int32),
        pltpu.VMEM((_CH, _DW), jnp.float32),
        pltpu.VMEM_SHARED((_NP, _DW), jnp.float32),
        pltpu.SemaphoreType.DMA,
    ],
    compiler_params=pltpu.CompilerParams(use_tc_tiling_on_sc=False),
)
def _deg_kernel(idx_hbm, ones_hbm, zeros_hbm, out_hbm, idx_v, ones_v, deg_sp,
                dsem):
    c = lax.axis_index("c")
    s = lax.axis_index("s")
    pltpu.sync_copy(ones_hbm, ones_v)
    pltpu.sync_copy(zeros_hbm, deg_sp.at[pl.ds(s * _RPT, _RPT)])
    pltpu.sync_copy(idx_hbm.at[c, s], idx_v)
    plsc.subcore_barrier()
    def body(j, carry):
        pltpu.async_copy(ones_v, deg_sp.at[idx_v.at[j]], dsem, add=True)
        return carry
    lax.fori_loop(0, _NCH, body, 0)
    def drain(j, carry):
        pltpu.make_async_copy(ones_v, deg_sp.at[idx_v.at[0]], dsem).wait()
        return carry
    lax.fori_loop(0, _NCH, drain, 0)
    plsc.subcore_barrier()
    pltpu.sync_copy(
        deg_sp.at[pl.ds(s * _RPT, _RPT)], out_hbm.at[c, pl.ds(s * _RPT, _RPT)]
    )


@functools.partial(
    pl.kernel,
    out_type=jax.ShapeDtypeStruct((_NC, _NP, _HALF), jnp.float32),
    mesh=_mesh,
    scratch_types=[
        pltpu.VMEM((_G, _CH), jnp.int32),
        pltpu.VMEM((_G, _CH), jnp.int32),
        [pltpu.VMEM((_CH, _HALF), jnp.float32) for _ in range(_NBUF)],
        pltpu.VMEM_SHARED((_NP, _HALF), jnp.float32),
        pltpu.VMEM_SHARED((_NP, _HALF), jnp.float32),
        pltpu.SemaphoreType.DMA,
        pltpu.SemaphoreType.DMA,
    ],
    compiler_params=pltpu.CompilerParams(use_tc_tiling_on_sc=False),
)
def _edge_kernel(hs_hbm, idx_hbm, zeros_hbm, out_hbm,
                 src_v, dst_v, bufs, hs_sp, acc_sp, gsem, ssem):
    c = lax.axis_index("c")
    s = lax.axis_index("s")
    r0 = s * _RPT
    pltpu.sync_copy(zeros_hbm, acc_sp.at[pl.ds(r0, _RPT)])
    pltpu.sync_copy(hs_hbm.at[c, pl.ds(r0, _RPT)], hs_sp.at[pl.ds(r0, _RPT)])
    plsc.subcore_barrier()
    def group(g, carry):
        pltpu.sync_copy(idx_hbm.at[0, s, pl.ds(g * _G, _G)], src_v)
        pltpu.sync_copy(idx_hbm.at[1, s, pl.ds(g * _G, _G)], dst_v)
        def round_(r, carry2):
            base = r * _NBUF
            def fire(b):
                pltpu.async_copy(hs_sp.at[src_v.at[base + b]], bufs[b], gsem)
            @pl.when(r > 0)
            def _():
                for b in range(_NBUF):
                    pltpu.make_async_copy(
                        bufs[b], acc_sp.at[dst_v.at[b]], ssem
                    ).wait()
                    fire(b)
            @pl.when(r == 0)
            def _():
                for b in range(_NBUF):
                    fire(b)
            for b in range(_NBUF):
                pltpu.make_async_copy(
                    hs_sp.at[src_v.at[base + b]], bufs[b], gsem
                ).wait()
                pltpu.async_copy(
                    bufs[b], acc_sp.at[dst_v.at[base + b]], ssem, add=True
                )
            return carry2
        carry = lax.fori_loop(0, _G // _NBUF, round_, carry)
        for b in range(_NBUF):
            pltpu.make_async_copy(
                bufs[b], acc_sp.at[dst_v.at[b]], ssem
            ).wait()
        return carry
    lax.fori_loop(0, _NCH // _G, group, 0)
    plsc.subcore_barrier()
    pltpu.sync_copy(
        acc_sp.at[pl.ds(r0, _RPT)],
        out_hbm.at[c, pl.ds(r0, _RPT)],
    )


_BLK1 = 1024


def _ln_body(x_ref, g_ref, b_ref, h_ref):
    xb = x_ref[...]
    mu = jnp.mean(xb, axis=-1, keepdims=True)
    xc = xb - mu
    var = jnp.mean(xc * xc, axis=-1, keepdims=True)
    h_ref[...] = xc * lax.rsqrt(var + _EPS) * g_ref[...] + b_ref[...]


_ln_call = pl.pallas_call(
    _ln_body,
    grid=(_NP // _BLK1,),
    in_specs=[
        pl.BlockSpec((_BLK1, _D), lambda i: (i, 0)),
        pl.BlockSpec((1, _D), lambda i: (0, 0)),
        pl.BlockSpec((1, _D), lambda i: (0, 0)),
    ],
    out_specs=pl.BlockSpec((_BLK1, _D), lambda i: (i, 0)),
    out_shape=jax.ShapeDtypeStruct((_NP, _D), jnp.float32),
)


def _hs_body(h_ref, deg_ref, hs_ref):
    ns = lax.rsqrt(jnp.maximum(deg_ref[0, :, :1], 1.0))
    rows = lax.broadcasted_iota(jnp.int32, (_BLK1, 1), 0) + pl.program_id(0) * _BLK1
    hs = jnp.where(rows < _N, h_ref[...] * ns, 0.0)
    hs_ref[...] = jnp.stack([hs[:, :_HALF], hs[:, _HALF:]], axis=0)


_hs_call = pl.pallas_call(
    _hs_body,
    grid=(_NP // _BLK1,),
    in_specs=[
        pl.BlockSpec((_BLK1, _D), lambda i: (i, 0)),
        pl.BlockSpec((1, _BLK1, _DW), lambda i: (0, i, 0)),
    ],
    out_specs=pl.BlockSpec((_NC, _BLK1, _HALF), lambda i: (0, i, 0)),
    out_shape=jax.ShapeDtypeStruct((_NC, _NP, _HALF), jnp.float32),
)


_BLK2 = 2000


def _ffna_body(h_ref, w_ref, b_ref, o_ref):
    dn = (((1,), (1,)), ((), ()))
    o = lax.dot_general(h_ref[...], w_ref[...], dn,
                        preferred_element_type=jnp.float32)
    o_ref[...] = o + b_ref[...]


_ffna_call = pl.pallas_call(
    _ffna_body,
    grid=(_N // _BLK2,),
    in_specs=[
        pl.BlockSpec((_BLK2, _D), lambda i: (i, 0)),
        pl.BlockSpec((_OUT, _D), lambda i: (0, 0)),
        pl.BlockSpec((1, _OUT), lambda i: (0, 0)),
    ],
    out_specs=pl.BlockSpec((_BLK2, _OUT), lambda i: (i, 0)),
    out_shape=jax.ShapeDtypeStruct((_N, _OUT), jnp.float32),
)


def _ffnb_body(p_ref, acc0_ref, acc1_ref, deg_ref, w_ref, o_ref):
    nd = lax.rsqrt(jnp.maximum(deg_ref[0, :, :1], 1.0))
    msg = jnp.concatenate([acc0_ref[...], acc1_ref[...]], axis=1) * nd
    dn = (((1,), (1,)), ((), ()))
    o = lax.dot_general(msg, w_ref[...], dn,
                        preferred_element_type=jnp.float32)
    o_ref[...] = o + p_ref[...]


_ffnb_call = pl.pallas_call(
    _ffnb_body,
    grid=(_N // _BLK2,),
    in_specs=[
        pl.BlockSpec((_BLK2, _OUT), lambda i: (i, 0)),
        pl.BlockSpec((_BLK2, _HALF), lambda i: (i, 0)),
        pl.BlockSpec((_BLK2, _HALF), lambda i: (i, 0)),
        pl.BlockSpec((1, _BLK2, _DW), lambda i: (1, i, 0)),
        pl.BlockSpec((_OUT, _D), lambda i: (0, 0)),
    ],
    out_specs=pl.BlockSpec((_BLK2, _OUT), lambda i: (i, 0)),
    out_shape=jax.ShapeDtypeStruct((_N, _OUT), jnp.float32),
)


def kernel(x, edge_index, gamma, beta, W, b):
    x_pad = jnp.concatenate(
        [x, jnp.zeros((_NP - _N, _D), jnp.float32)], axis=0
    )
    pad = jnp.full((2, _EP - _E), _NP - 1, jnp.int32)
    ei = jnp.concatenate([edge_index, pad], axis=1).reshape(2, _NS, _NCH, _CH)
    deg = _deg_kernel(
        ei,
        jnp.ones((_CH, _DW), jnp.float32),
        jnp.zeros((_RPT, _DW), jnp.float32),
    )
    h = _ln_call(x_pad, gamma.reshape(1, _D), beta.reshape(1, _D))
    hs = _hs_call(h, deg)
    zeros_tile = jnp.zeros((_RPT, _HALF), jnp.float32)
    acc = _edge_kernel(hs, ei, zeros_tile)
    part = _ffna_call(h[:_N], W[:, :_D], b.reshape(1, _OUT))
    out = _ffnb_call(part, acc[0, :_N], acc[1, :_N], deg, W[:, _D:])
    return out


# speedup vs baseline: 1.5188x; 1.0092x over previous
import functools

import jax
import jax.numpy as jnp
from jax import lax
from jax.experimental import pallas as pl
from jax.experimental.pallas import tpu as pltpu
from jax.experimental.pallas import tpu_sc as plsc

_N = 10000
_E = 320000
_D = 128
_OUT = 128
_EPS = 1e-5

_NC = 2
_NS = 16
_NP = 10240
_RPT = _NP // _NS
_CH = 128
_NCH = 160
_NBUF = 5
_G = 20
_HALF = _D // _NC
_EPT = _NCH * _CH
_EP = _EPT * _NS

_mesh = plsc.VectorSubcoreMesh(
    core_axis_name="c", subcore_axis_name="s", num_cores=_NC, num_subcores=_NS
)


_DW = 16


@functools.partial(
    pl.kernel,
    out_type=jax.ShapeDtypeStruct((_NC, _NP, _DW), jnp.float32),
    mesh=_mesh,
    scratch_types=[
        pltpu.VMEM((_NCH, _CH), jnp.int32),
        pltpu.VMEM((_CH, _DW), jnp.float32),
        pltpu.VMEM_SHARED((_NP, _DW), jnp.float32),
        pltpu.SemaphoreType.DMA,
    ],
    compiler_params=pltpu.CompilerParams(use_tc_tiling_on_sc=False),
)
def _deg_kernel(idx_hbm, ones_hbm, zeros_hbm, out_hbm, idx_v, ones_v, deg_sp,
                dsem):
    c = lax.axis_index("c")
    s = lax.axis_index("s")
    pltpu.sync_copy(ones_hbm, ones_v)
    pltpu.sync_copy(zeros_hbm, deg_sp.at[pl.ds(s * _RPT, _RPT)])
    pltpu.sync_copy(idx_hbm.at[c, s], idx_v)
    plsc.subcore_barrier()
    def body(j, carry):
        pltpu.async_copy(ones_v, deg_sp.at[idx_v.at[j]], dsem, add=True)
        return carry
    lax.fori_loop(0, _NCH, body, 0)
    def drain(j, carry):
        pltpu.make_async_copy(ones_v, deg_sp.at[idx_v.at[0]], dsem).wait()
        return carry
    lax.fori_loop(0, _NCH, drain, 0)
    plsc.subcore_barrier()
    pltpu.sync_copy(
        deg_sp.at[pl.ds(s * _RPT, _RPT)], out_hbm.at[c, pl.ds(s * _RPT, _RPT)]
    )


@functools.partial(
    pl.kernel,
    out_type=jax.ShapeDtypeStruct((_NC, _NP, _HALF), jnp.float32),
    mesh=_mesh,
    scratch_types=[
        pltpu.VMEM((_G, _CH), jnp.int32),
        pltpu.VMEM((_G, _CH), jnp.int32),
        [pltpu.VMEM((_CH, _HALF), jnp.float32) for _ in range(_NBUF)],
        pltpu.VMEM_SHARED((_NP, _HALF), jnp.float32),
        pltpu.VMEM_SHARED((_NP, _HALF), jnp.float32),
        pltpu.SemaphoreType.DMA,
        pltpu.SemaphoreType.DMA,
    ],
    compiler_params=pltpu.CompilerParams(use_tc_tiling_on_sc=False),
)
def _edge_kernel(hs_hbm, idx_hbm, zeros_hbm, out_hbm,
                 src_v, dst_v, bufs, hs_sp, acc_sp, gsem, ssem):
    c = lax.axis_index("c")
    s = lax.axis_index("s")
    r0 = s * _RPT
    pltpu.sync_copy(zeros_hbm, acc_sp.at[pl.ds(r0, _RPT)])
    pltpu.sync_copy(hs_hbm.at[c, pl.ds(r0, _RPT)], hs_sp.at[pl.ds(r0, _RPT)])
    plsc.subcore_barrier()
    def group(g, carry):
        pltpu.sync_copy(idx_hbm.at[0, s, pl.ds(g * _G, _G)], src_v)
        pltpu.sync_copy(idx_hbm.at[1, s, pl.ds(g * _G, _G)], dst_v)
        def round_(r, carry2):
            base = r * _NBUF
            def fire(b):
                pltpu.async_copy(hs_sp.at[src_v.at[base + b]], bufs[b], gsem)
            @pl.when(r > 0)
            def _():
                for b in range(_NBUF):
                    pltpu.make_async_copy(
                        bufs[b], acc_sp.at[dst_v.at[b]], ssem
                    ).wait()
                    fire(b)
            @pl.when(r == 0)
            def _():
                for b in range(_NBUF):
                    fire(b)
            for b in range(_NBUF):
                pltpu.make_async_copy(
                    hs_sp.at[src_v.at[base + b]], bufs[b], gsem
                ).wait()
                pltpu.async_copy(
                    bufs[b], acc_sp.at[dst_v.at[base + b]], ssem, add=True
                )
            return carry2
        carry = lax.fori_loop(0, _G // _NBUF, round_, carry)
        for b in range(_NBUF):
            pltpu.make_async_copy(
                bufs[b], acc_sp.at[dst_v.at[b]], ssem
            ).wait()
        return carry
    lax.fori_loop(0, _NCH // _G, group, 0)
    plsc.subcore_barrier()
    pltpu.sync_copy(
        acc_sp.at[pl.ds(r0, _RPT)],
        out_hbm.at[c, pl.ds(r0, _RPT)],
    )


_BLK1 = 1024


def _ln_body(x_ref, g_ref, b_ref, h_ref):
    xb = x_ref[...]
    mu = jnp.mean(xb, axis=-1, keepdims=True)
    xc = xb - mu
    var = jnp.mean(xc * xc, axis=-1, keepdims=True)
    h_ref[...] = xc * lax.rsqrt(var + _EPS) * g_ref[...] + b_ref[...]


_ln_call = pl.pallas_call(
    _ln_body,
    grid=(_NP // _BLK1,),
    in_specs=[
        pl.BlockSpec((_BLK1, _D), lambda i: (i, 0)),
        pl.BlockSpec((1, _D), lambda i: (0, 0)),
        pl.BlockSpec((1, _D), lambda i: (0, 0)),
    ],
    out_specs=pl.BlockSpec((_BLK1, _D), lambda i: (i, 0)),
    out_shape=jax.ShapeDtypeStruct((_NP, _D), jnp.float32),
)


def _hs_body(h_ref, deg_ref, hs_ref):
    ns = lax.rsqrt(jnp.maximum(deg_ref[0, :, :1], 1.0))
    rows = lax.broadcasted_iota(jnp.int32, (_BLK1, 1), 0) + pl.program_id(0) * _BLK1
    hs = jnp.where(rows < _N, h_ref[...] * ns, 0.0)
    hs_ref[...] = jnp.stack([hs[:, :_HALF], hs[:, _HALF:]], axis=0)


_hs_call = pl.pallas_call(
    _hs_body,
    grid=(_NP // _BLK1,),
    in_specs=[
        pl.BlockSpec((_BLK1, _D), lambda i: (i, 0)),
        pl.BlockSpec((1, _BLK1, _DW), lambda i: (0, i, 0)),
    ],
    out_specs=pl.BlockSpec((_NC, _BLK1, _HALF), lambda i: (0, i, 0)),
    out_shape=jax.ShapeDtypeStruct((_NC, _NP, _HALF), jnp.float32),
)


_BLK2 = 2000


def _ffn_body(h_ref, acc0_ref, acc1_ref, deg_ref, w_ref, b_ref, o_ref):
    nd = lax.rsqrt(jnp.maximum(deg_ref[0, :, :1], 1.0))
    msg = jnp.concatenate([acc0_ref[...], acc1_ref[...]], axis=1) * nd
    w = w_ref[...]
    dn = (((1,), (1,)), ((), ()))
    o = lax.dot_general(h_ref[...], w[:, :_D], dn, preferred_element_type=jnp.float32)
    o = o + lax.dot_general(msg, w[:, _D:], dn, preferred_element_type=jnp.float32)
    o_ref[...] = o + b_ref[...]


_ffn_call = pl.pallas_call(
    _ffn_body,
    grid=(_N // _BLK2,),
    in_specs=[
        pl.BlockSpec((_BLK2, _D), lambda i: (i, 0)),
        pl.BlockSpec((_BLK2, _HALF), lambda i: (i, 0)),
        pl.BlockSpec((_BLK2, _HALF), lambda i: (i, 0)),
        pl.BlockSpec((1, _BLK2, _DW), lambda i: (1, i, 0)),
        pl.BlockSpec((_OUT, 2 * _D), lambda i: (0, 0)),
        pl.BlockSpec((1, _OUT), lambda i: (0, 0)),
    ],
    out_specs=pl.BlockSpec((_BLK2, _OUT), lambda i: (i, 0)),
    out_shape=jax.ShapeDtypeStruct((_N, _OUT), jnp.float32),
)


def kernel(x, edge_index, gamma, beta, W, b):
    x_pad = jnp.concatenate(
        [x, jnp.zeros((_NP - _N, _D), jnp.float32)], axis=0
    )
    pad = jnp.full((2, _EP - _E), _NP - 1, jnp.int32)
    ei = jnp.concatenate([edge_index, pad], axis=1).reshape(2, _NS, _NCH, _CH)
    deg = _deg_kernel(
        ei,
        jnp.ones((_CH, _DW), jnp.float32),
        jnp.zeros((_RPT, _DW), jnp.float32),
    )
    h = _ln_call(x_pad, gamma.reshape(1, _D), beta.reshape(1, _D))
    hs = _hs_call(h, deg)
    zeros_tile = jnp.zeros((_RPT, _HALF), jnp.float32)
    acc = _edge_kernel(hs, ei, zeros_tile)
    out = _ffn_call(
        h[:_N], acc[0, :_N], acc[1, :_N], deg, W, b.reshape(1, _OUT),
    )
    return out
